# baseline (speedup 1.0000x reference)
import jax
import jax.numpy as jnp
from jax.experimental import pallas as pl

_EPS = 1e-12
_BLOCK_S = 1024


def _ln_add_kernel(x_ref, pos_ref, gamma_ref, beta_ref, out_ref):
    j = pl.program_id(0)
    x = x_ref[...]
    p = pos_ref[pl.ds(j * _BLOCK_S, _BLOCK_S), :]
    e = x + p[None, :, :]
    mean = jnp.mean(e, axis=-1, keepdims=True)
    c = e - mean
    var = jnp.mean(c * c, axis=-1, keepdims=True)
    inv = jax.lax.rsqrt(var + _EPS)
    out_ref[...] = c * inv * gamma_ref[...][None] + beta_ref[...][None]


def kernel(inputs_embeds, pos_table, ln_gamma, ln_beta):
    B, S, H = inputs_embeds.shape
    bs = _BLOCK_S
    grid = (S // bs, B)
    return pl.pallas_call(
        _ln_add_kernel,
        grid=grid,
        in_specs=[
            pl.BlockSpec((1, bs, H), lambda j, b: (b, j, 0)),
            pl.BlockSpec((S, H), lambda j, b: (0, 0)),
            pl.BlockSpec((1, H), lambda j, b: (0, 0)),
            pl.BlockSpec((1, H), lambda j, b: (0, 0)),
        ],
        out_specs=pl.BlockSpec((1, bs, H), lambda j, b: (b, j, 0)),
        out_shape=jax.ShapeDtypeStruct((B, S, H), jnp.float32),
    )(inputs_embeds, pos_table, ln_gamma.reshape(1, H), ln_beta.reshape(1, H))
